# baseline (speedup 1.0000x reference)
import jax
import jax.numpy as jnp
from jax.experimental import pallas as pl
from jax.experimental.pallas import tpu as pltpu

U = 10000
I = 10000
G = 1000
D = 128

MB = 200
NM = G // MB
SB = 5
RB = 600
NR = (U + I + G) // (SB * RB)


def _msg_kernel(uh_ref, ih_ref, ue_ref, ie_ref, wt_ref, b_ref, msg_ref):
    bf = jnp.bfloat16
    u_msg = jnp.dot(uh_ref[...].astype(bf), ue_ref[...].astype(bf),
                    preferred_element_type=jnp.float32)
    i_msg = jnp.dot(ih_ref[...].astype(bf), ie_ref[...].astype(bf),
                    preferred_element_type=jnp.float32)
    msg_ref[...] = (
        jnp.dot(u_msg, wt_ref[:D, :], preferred_element_type=jnp.float32)
        + jnp.dot(i_msg, wt_ref[D:, :], preferred_element_type=jnp.float32)
        + b_ref[...]
    )


def _norm_kernel(*refs):
    fh_refs = refs[:SB]
    msg_ref = refs[SB]
    out_ref = refs[SB + 1]
    bf = jnp.bfloat16
    msg_bf = msg_ref[...].astype(bf)
    for s in range(SB):
        out_ref[s * RB:(s + 1) * RB, :] = jnp.dot(
            fh_refs[s][...].astype(bf), msg_bf,
            preferred_element_type=jnp.float32)


def kernel(user_emb, item_emb, group_emb, user_hyper_graph,
           item_hyper_graph, full_hyper, W_agg, b_agg):
    wt = W_agg.T
    b2 = b_agg.reshape(1, D)

    msg = pl.pallas_call(
        _msg_kernel,
        grid=(NM,),
        in_specs=[
            pl.BlockSpec((MB, U), lambda m: (m, 0)),
            pl.BlockSpec((MB, I), lambda m: (m, 0)),
            pl.BlockSpec((U, D), lambda m: (0, 0)),
            pl.BlockSpec((I, D), lambda m: (0, 0)),
            pl.BlockSpec((2 * D, D), lambda m: (0, 0)),
            pl.BlockSpec((1, D), lambda m: (0, 0)),
        ],
        out_specs=pl.BlockSpec((MB, D), lambda m: (m, 0)),
        out_shape=jax.ShapeDtypeStruct((G, D), jnp.float32),
        compiler_params=pltpu.CompilerParams(
            dimension_semantics=("arbitrary",)),
    )(user_hyper_graph, item_hyper_graph, user_emb, item_emb, wt, b2)

    fh_specs = [
        pl.BlockSpec((RB, G), lambda r, s=s: (r * SB + s, 0))
        for s in range(SB)
    ]
    norm_emb = pl.pallas_call(
        _norm_kernel,
        grid=(NR,),
        in_specs=fh_specs + [pl.BlockSpec((G, D), lambda r: (0, 0))],
        out_specs=pl.BlockSpec((SB * RB, D), lambda r: (r, 0)),
        out_shape=jax.ShapeDtypeStruct((U + I + G, D), jnp.float32),
        compiler_params=pltpu.CompilerParams(
            dimension_semantics=("arbitrary",)),
    )(*([full_hyper] * SB), msg)

    return (norm_emb, msg)


# speedup vs baseline: 1.5480x; 1.5480x over previous
import jax
import jax.numpy as jnp
from jax.experimental import pallas as pl
from jax.experimental.pallas import tpu as pltpu

U = 10000
I = 10000
G = 1000
D = 128
N = U + I + G

MB = 200
NM = G // MB
CB = 4096
NC = (N + CB - 1) // CB


def _msg_kernel(uh_ref, ih_ref, ue_ref, ie_ref, wt_ref, b_ref, msg_ref):
    bf = jnp.bfloat16
    u_msg = jnp.dot(uh_ref[...].astype(bf), ue_ref[...].astype(bf),
                    preferred_element_type=jnp.float32)
    i_msg = jnp.dot(ih_ref[...].astype(bf), ie_ref[...].astype(bf),
                    preferred_element_type=jnp.float32)
    msg_ref[...] = (
        jnp.dot(u_msg, wt_ref[:D, :], preferred_element_type=jnp.float32)
        + jnp.dot(i_msg, wt_ref[D:, :], preferred_element_type=jnp.float32)
        + b_ref[...]
    )


def _norm_kernel(fht_ref, msg_ref, out_ref):
    bf = jnp.bfloat16
    out_ref[...] = jax.lax.dot_general(
        msg_ref[...].astype(bf), fht_ref[...].astype(bf),
        (((0,), (0,)), ((), ())),
        preferred_element_type=jnp.float32)


def kernel(user_emb, item_emb, group_emb, user_hyper_graph,
           item_hyper_graph, full_hyper, W_agg, b_agg):
    wt = W_agg.T
    b2 = b_agg.reshape(1, D)
    fh_t = full_hyper.T

    msg = pl.pallas_call(
        _msg_kernel,
        grid=(NM,),
        in_specs=[
            pl.BlockSpec((MB, U), lambda m: (m, 0)),
            pl.BlockSpec((MB, I), lambda m: (m, 0)),
            pl.BlockSpec((U, D), lambda m: (0, 0)),
            pl.BlockSpec((I, D), lambda m: (0, 0)),
            pl.BlockSpec((2 * D, D), lambda m: (0, 0)),
            pl.BlockSpec((1, D), lambda m: (0, 0)),
        ],
        out_specs=pl.BlockSpec((MB, D), lambda m: (m, 0)),
        out_shape=jax.ShapeDtypeStruct((G, D), jnp.float32),
        compiler_params=pltpu.CompilerParams(
            dimension_semantics=("arbitrary",)),
    )(user_hyper_graph, item_hyper_graph, user_emb, item_emb, wt, b2)

    norm_t = pl.pallas_call(
        _norm_kernel,
        grid=(NC,),
        in_specs=[
            pl.BlockSpec((G, CB), lambda c: (0, c)),
            pl.BlockSpec((G, D), lambda c: (0, 0)),
        ],
        out_specs=pl.BlockSpec((D, CB), lambda c: (0, c)),
        out_shape=jax.ShapeDtypeStruct((D, N), jnp.float32),
        compiler_params=pltpu.CompilerParams(
            dimension_semantics=("arbitrary",)),
    )(fh_t, msg)

    return (norm_t.T, msg)


# speedup vs baseline: 2.0902x; 1.3503x over previous
import jax
import jax.numpy as jnp
from jax.experimental import pallas as pl
from jax.experimental.pallas import tpu as pltpu

U = 10000
I = 10000
G = 1000
D = 128
N = U + I + G

MB = 200
NM = G // MB
CB = 4096
NC = (N + CB - 1) // CB


def _msg_kernel(uh_ref, ih_ref, ue_ref, ie_ref, wt_ref, b_ref, msg_ref):
    bf = jnp.bfloat16
    u_msg = jnp.dot(uh_ref[...].astype(bf), ue_ref[...].astype(bf),
                    preferred_element_type=jnp.float32)
    i_msg = jnp.dot(ih_ref[...].astype(bf), ie_ref[...].astype(bf),
                    preferred_element_type=jnp.float32)
    msg_ref[...] = (
        jnp.dot(u_msg, wt_ref[:D, :], preferred_element_type=jnp.float32)
        + jnp.dot(i_msg, wt_ref[D:, :], preferred_element_type=jnp.float32)
        + b_ref[...]
    )


def _norm_kernel(fht_ref, msg_ref, out_ref):
    bf = jnp.bfloat16
    out_ref[...] = jax.lax.dot_general(
        fht_ref[...].astype(bf), msg_ref[...].astype(bf),
        (((0,), (0,)), ((), ())),
        preferred_element_type=jnp.float32)


def kernel(user_emb, item_emb, group_emb, user_hyper_graph,
           item_hyper_graph, full_hyper, W_agg, b_agg):
    wt = W_agg.T
    b2 = b_agg.reshape(1, D)
    fh_t = full_hyper.T

    msg = pl.pallas_call(
        _msg_kernel,
        grid=(NM,),
        in_specs=[
            pl.BlockSpec((MB, U), lambda m: (m, 0)),
            pl.BlockSpec((MB, I), lambda m: (m, 0)),
            pl.BlockSpec((U, D), lambda m: (0, 0)),
            pl.BlockSpec((I, D), lambda m: (0, 0)),
            pl.BlockSpec((2 * D, D), lambda m: (0, 0)),
            pl.BlockSpec((1, D), lambda m: (0, 0)),
        ],
        out_specs=pl.BlockSpec((MB, D), lambda m: (m, 0)),
        out_shape=jax.ShapeDtypeStruct((G, D), jnp.float32),
        compiler_params=pltpu.CompilerParams(
            dimension_semantics=("arbitrary",)),
    )(user_hyper_graph, item_hyper_graph, user_emb, item_emb, wt, b2)

    norm_t = pl.pallas_call(
        _norm_kernel,
        grid=(NC,),
        in_specs=[
            pl.BlockSpec((G, CB), lambda c: (0, c)),
            pl.BlockSpec((G, D), lambda c: (0, 0)),
        ],
        out_specs=pl.BlockSpec((CB, D), lambda c: (c, 0)),
        out_shape=jax.ShapeDtypeStruct((N, D), jnp.float32),
        compiler_params=pltpu.CompilerParams(
            dimension_semantics=("arbitrary",)),
    )(fh_t, msg)

    return (norm_t, msg)
